# baseline (speedup 1.0000x reference)
import jax
import jax.numpy as jnp
from jax import lax
from jax.experimental import pallas as pl
from jax.experimental.pallas import tpu as pltpu
from jax.experimental.pallas import tpu_sc as plsc

SENT_LEN = 50
BATCH = 1024
MAX_WORD_LEN = 20
CHAR_VOCAB = 262
CHAR_EMBED = 20
N_WORDS = SENT_LEN * BATCH
PADW = 24
TABN = CHAR_VOCAB * CHAR_EMBED

NUM_CORES = 2
NUM_SUBCORES = 16
NW = NUM_CORES * NUM_SUBCORES
SW = N_WORDS // NW
C = 64
NCH = SW // C
WBLK = MAX_WORD_LEN * CHAR_EMBED
CE = C * WBLK


def _sc_body(inp_ref, w2c_ref, tab_ref, out_e_ref, out_l_ref,
             widx, spell, lens, tab_v, ebuf, sem_s):
  cid = lax.axis_index("c")
  sid = lax.axis_index("s")
  wid = sid * NUM_CORES + cid
  wbase = wid * SW

  pltpu.sync_copy(tab_ref, tab_v)

  def chunk_body(ci, carry):
    cb = wbase + ci * C
    pltpu.sync_copy(inp_ref.at[pl.ds(cb, C)], widx)
    pltpu.async_copy(w2c_ref.at[widx], spell, sem_s).wait()

    def extract_len(i, carry2):
      lane = lax.iota(jnp.int32, 16)
      wj = i * 16 + lane
      col_len = jnp.full((16,), MAX_WORD_LEN, jnp.int32)
      lv = plsc.load_gather(spell, [wj, col_len])
      lens[pl.ds(ci * C + i * 16, 16)] = lv
      return carry2

    lax.fori_loop(0, C // 16, extract_len, 0)

    def build(i, carry2):
      lane = lax.iota(jnp.int32, 16)
      e = i * 16 + lane
      t = lax.shift_right_logical(e, 2)
      jw = lax.shift_right_logical(t * 5243, 19)
      r = e - jw * WBLK
      cpos = lax.shift_right_logical(r * 3277, 16)
      d = r - cpos * CHAR_EMBED
      chars = plsc.load_gather(spell, [jw, cpos])
      vals = plsc.load_gather(tab_v, [chars * CHAR_EMBED + d])
      ebuf[pl.ds(i * 16, 16)] = vals
      return carry2

    lax.fori_loop(0, CE // 16, build, 0, unroll=8)

    pltpu.sync_copy(ebuf, out_e_ref.at[pl.ds(cb * WBLK, CE)])
    return carry

  lax.fori_loop(0, NCH, chunk_body, 0)
  pltpu.sync_copy(lens, out_l_ref.at[pl.ds(wbase, SW)])


@jax.jit
def _char2vec(inp_flat, w2c_pad, tab_flat):
  mesh = plsc.VectorSubcoreMesh(
      core_axis_name="c", subcore_axis_name="s",
      num_cores=NUM_CORES, num_subcores=NUM_SUBCORES)
  k = pl.kernel(
      _sc_body,
      out_type=[
          jax.ShapeDtypeStruct((N_WORDS * WBLK,), jnp.float32),
          jax.ShapeDtypeStruct((N_WORDS,), jnp.int32),
      ],
      mesh=mesh,
      scratch_types=[
          pltpu.VMEM((C,), jnp.int32),
          pltpu.VMEM((C, PADW), jnp.int32),
          pltpu.VMEM((SW,), jnp.int32),
          pltpu.VMEM((TABN,), jnp.float32),
          pltpu.VMEM((CE,), jnp.float32),
          pltpu.SemaphoreType.DMA,
      ],
      compiler_params=pltpu.CompilerParams(
          use_tc_tiling_on_sc=False, needs_layout_passes=False),
  )
  return k(inp_flat, w2c_pad, tab_flat)


def kernel(inp, word2chars, charEmbedTable):
  sent_len, batch, _ = inp.shape
  inp_flat = inp.reshape(-1)
  w2c_pad = jnp.pad(word2chars, ((0, 0), (0, PADW - MAX_WORD_LEN - 1)))
  tab_flat = charEmbedTable.reshape(-1)
  char_embeds, len_flat = _char2vec(inp_flat, w2c_pad, tab_flat)
  char_embeds = char_embeds.reshape(sent_len * batch, MAX_WORD_LEN,
                                    CHAR_EMBED)
  return (char_embeds, len_flat)


# speedup vs baseline: 3.1468x; 3.1468x over previous
import jax
import jax.numpy as jnp
from jax import lax
from jax.experimental import pallas as pl
from jax.experimental.pallas import tpu as pltpu
from jax.experimental.pallas import tpu_sc as plsc

SENT_LEN = 50
BATCH = 1024
MAX_WORD_LEN = 20
CHAR_VOCAB = 262
CHAR_EMBED = 20
DPAD = 24
N_WORDS = SENT_LEN * BATCH
PADW = 24
TABN = CHAR_VOCAB * CHAR_EMBED

NUM_CORES = 2
NUM_SUBCORES = 16
NW = NUM_CORES * NUM_SUBCORES
SW = N_WORDS // NW
C = 80
NCH = SW // C
NWG = C // 16


def _sc_body(inp_ref, w2c_ref, tab_ref, out_e_ref, out_l_ref,
             widx, spell, lens, tab_v, ebuf, sem_s):
  cid = lax.axis_index("c")
  sid = lax.axis_index("s")
  wid = sid * NUM_CORES + cid
  wbase = wid * SW

  pltpu.sync_copy(tab_ref, tab_v)

  def chunk_body(ci, carry):
    cb = wbase + ci * C
    pltpu.sync_copy(inp_ref.at[pl.ds(cb, C)], widx)
    pltpu.async_copy(w2c_ref.at[widx], spell, sem_s).wait()

    def extract_len(i, carry2):
      lane = lax.iota(jnp.int32, 16)
      wj = i * 16 + lane
      col_len = jnp.full((16,), MAX_WORD_LEN, jnp.int32)
      lv = plsc.load_gather(spell, [wj, col_len])
      lens[pl.ds(ci * C + i * 16, 16)] = lv
      return carry2

    lax.fori_loop(0, NWG, extract_len, 0)

    def build_cg(cg, carry2):
      c = cg // NWG
      g = cg - c * NWG
      lane = lax.iota(jnp.int32, 16)
      wl = g * 16 + lane
      c_vec = jnp.full((16,), 0, jnp.int32) + c
      chars = plsc.load_gather(spell, [wl, c_vec])
      fb = chars * CHAR_EMBED

      def build_d(d, carry3):
        vals = plsc.load_gather(tab_v, [fb + d])
        ebuf[c, d, pl.ds(g * 16, 16)] = vals
        return carry3

      lax.fori_loop(0, CHAR_EMBED, build_d, 0, unroll=5)
      return carry2

    lax.fori_loop(0, MAX_WORD_LEN * NWG, build_cg, 0)

    pltpu.sync_copy(ebuf, out_e_ref.at[:, :, pl.ds(cb, C)])
    return carry

  lax.fori_loop(0, NCH, chunk_body, 0)
  pltpu.sync_copy(lens, out_l_ref.at[pl.ds(wbase, SW)])


@jax.jit
def _char2vec(inp_flat, w2c_pad, tab_flat):
  mesh = plsc.VectorSubcoreMesh(
      core_axis_name="c", subcore_axis_name="s",
      num_cores=NUM_CORES, num_subcores=NUM_SUBCORES)
  k = pl.kernel(
      _sc_body,
      out_type=[
          jax.ShapeDtypeStruct((MAX_WORD_LEN, DPAD, N_WORDS), jnp.float32),
          jax.ShapeDtypeStruct((N_WORDS,), jnp.int32),
      ],
      mesh=mesh,
      scratch_types=[
          pltpu.VMEM((C,), jnp.int32),
          pltpu.VMEM((C, PADW), jnp.int32),
          pltpu.VMEM((SW,), jnp.int32),
          pltpu.VMEM((TABN,), jnp.float32),
          pltpu.VMEM((MAX_WORD_LEN, DPAD, C), jnp.float32),
          pltpu.SemaphoreType.DMA,
      ],
      compiler_params=pltpu.CompilerParams(
          use_tc_tiling_on_sc=False, needs_layout_passes=False),
  )
  return k(inp_flat, w2c_pad, tab_flat)


def kernel(inp, word2chars, charEmbedTable):
  sent_len, batch, _ = inp.shape
  inp_flat = inp.reshape(-1)
  w2c_pad = jnp.pad(word2chars, ((0, 0), (0, PADW - MAX_WORD_LEN - 1)))
  tab_flat = charEmbedTable.reshape(-1)
  emb_t, len_flat = _char2vec(inp_flat, w2c_pad, tab_flat)
  char_embeds = jnp.transpose(emb_t[:, :CHAR_EMBED, :], (2, 0, 1))
  return (char_embeds, len_flat)
